# baseline (speedup 1.0000x reference)
import functools

import jax
import jax.numpy as jnp
from jax.experimental import pallas as pl
from jax.experimental.pallas import tpu as pltpu

_BLOCK = 2048
_BP = _BLOCK // 4
_OBS = 32
_ACT = 16
_H = 128


def _body(s_ref, a_ref, Ws_ref, Wa_ref, b1_ref, Wg_ref, bg_ref, W2_ref, b2_ref,
          Wq1a_ref, bq1a_ref, Wq1b_ref, bq1b_ref,
          Wq2a_ref, bq2a_ref, Wq2b_ref, bq2b_ref,
          q1_ref, q2_ref):
    f32 = jnp.float32
    bf16 = jnp.bfloat16
    acc = jnp.dot(s_ref[0], Ws_ref[0], preferred_element_type=f32)
    acc += jnp.dot(s_ref[1], Ws_ref[1], preferred_element_type=f32)
    acc += jnp.dot(s_ref[2], Ws_ref[2], preferred_element_type=f32)
    acc += jnp.dot(a_ref[0], Wa_ref[0], preferred_element_type=f32)
    acc += jnp.dot(a_ref[1], Wa_ref[1], preferred_element_type=f32)
    acc += jnp.dot(a_ref[2], Wa_ref[2], preferred_element_type=f32)
    F = jnp.maximum(acc + b1_ref[:], 0.0)
    Fb = F.astype(bf16)

    first = pl.program_id(0) == 0
    row = jax.lax.broadcasted_iota(jnp.int32, (_BP, 1), 0)

    xw = [jnp.dot(Fb[:, _H * j:_H * (j + 1)], Wg_ref[:],
                  preferred_element_type=f32) for j in range(4)]
    clique = (xw[0][0:1, :] + xw[1][0:1, :] + xw[2][0:1, :]) * (1.0 / 3.0)
    fix = jnp.logical_and(first, row < 1)
    h1cat = []
    h2cat = []
    for j in range(4):
        xwj = jnp.where(fix, clique, xw[j]) if j < 3 else xw[j]
        fj = F[:, _H * j:_H * (j + 1)]
        g = (jnp.maximum(xwj + bg_ref[:], 0.0) + fj).astype(bf16)
        fc2 = jnp.maximum(jnp.dot(g, W2_ref[:], preferred_element_type=f32)
                          + b2_ref[:], 0.0).astype(bf16)
        h1cat.append(jnp.maximum(
            jnp.dot(fc2, Wq1a_ref[:], preferred_element_type=f32)
            + bq1a_ref[:], 0.0).astype(bf16))
        h2cat.append(jnp.maximum(
            jnp.dot(fc2, Wq2a_ref[:], preferred_element_type=f32)
            + bq2a_ref[:], 0.0).astype(bf16))
    H1 = jnp.concatenate(h1cat, axis=1)
    H2 = jnp.concatenate(h2cat, axis=1)
    q1p = jnp.dot(H1, Wq1b_ref[:], preferred_element_type=f32) + bq1b_ref[:]
    q2p = jnp.dot(H2, Wq2b_ref[:], preferred_element_type=f32) + bq2b_ref[:]
    for j in range(4):
        q1_ref[pl.Slice(j, _BP, 4), :] = q1p[:, j:j + 1]
        q2_ref[pl.Slice(j, _BP, 4), :] = q2p[:, j:j + 1]


def _shift_weights(W1):
    eye4 = jnp.eye(4, dtype=W1.dtype)
    Ws = jnp.stack([jnp.kron(eye4, W1[_OBS * k:_OBS * (k + 1), :])
                    for k in range(3)])
    Wa = jnp.stack([jnp.kron(
        eye4, W1[3 * _OBS + _ACT * k:3 * _OBS + _ACT * (k + 1), :])
        for k in range(3)])
    return Ws, Wa


def _blockdiag_head(w):
    return jnp.kron(jnp.eye(4, dtype=w.dtype), w)


@functools.partial(jax.jit, static_argnames=("interpret",))
def kernel(s, a, W1, b1, Wg, bg, W2, b2, Wq1a, bq1a, Wq1b, bq1b,
           Wq2a, bq2a, Wq2b, bq2b, interpret=False):
    na, n, obs = s.shape
    act = a.shape[2]
    h = W1.shape[1]
    bf16 = jnp.bfloat16
    sp = s.astype(bf16).reshape(na, n // 4, 4 * obs)
    ap = a.astype(bf16).reshape(na, n // 4, 4 * act)
    Ws, Wa = _shift_weights(W1.astype(bf16))
    b1t = jnp.tile(b1.reshape(1, h), (1, 4))

    def rows(i):
        return (0, i, 0)

    def full(i):
        return (0, 0)

    def full3(i):
        return (0, 0, 0)

    in_specs = [
        pl.BlockSpec((na, _BP, 4 * obs), rows),
        pl.BlockSpec((na, _BP, 4 * act), rows),
        pl.BlockSpec(Ws.shape, full3),
        pl.BlockSpec(Wa.shape, full3),
        pl.BlockSpec((1, 4 * h), full),
        pl.BlockSpec(Wg.shape, full),
        pl.BlockSpec((1, h), full),
        pl.BlockSpec(W2.shape, full),
        pl.BlockSpec((1, h), full),
        pl.BlockSpec(Wq1a.shape, full),
        pl.BlockSpec((1, h), full),
        pl.BlockSpec((4 * h, 4), full),
        pl.BlockSpec((1, 1), full),
        pl.BlockSpec(Wq2a.shape, full),
        pl.BlockSpec((1, h), full),
        pl.BlockSpec((4 * h, 4), full),
        pl.BlockSpec((1, 1), full),
    ]
    out_specs = [
        pl.BlockSpec((_BLOCK, 1), lambda i: (i, 0)),
        pl.BlockSpec((_BLOCK, 1), lambda i: (i, 0)),
    ]
    q1, q2 = pl.pallas_call(
        _body,
        grid=(pl.cdiv(n, _BLOCK),),
        in_specs=in_specs,
        out_specs=out_specs,
        out_shape=[jax.ShapeDtypeStruct((n, 1), jnp.float32)] * 2,
        compiler_params=pltpu.CompilerParams(
            dimension_semantics=("parallel",)),
        interpret=interpret,
    )(sp, ap, Ws, Wa, b1t, Wg.astype(bf16), bg.reshape(1, h),
      W2.astype(bf16), b2.reshape(1, h),
      Wq1a.astype(bf16), bq1a.reshape(1, h),
      _blockdiag_head(Wq1b.astype(bf16)), bq1b.reshape(1, 1),
      Wq2a.astype(bf16), bq2a.reshape(1, h),
      _blockdiag_head(Wq2b.astype(bf16)), bq2b.reshape(1, 1))
    return (q1, q2)


# speedup vs baseline: 1.1352x; 1.1352x over previous
import functools

import jax
import jax.numpy as jnp
from jax.experimental import pallas as pl
from jax.experimental.pallas import tpu as pltpu

_BLOCK = 2000
_OBS = 32
_ACT = 16


def _body(s_ref, a_ref, W1_ref, b1_ref, Wg_ref, bg_ref, W2_ref, b2_ref,
          Wq1a_ref, bq1a_ref, Wq1b_ref, bq1b_ref,
          Wq2a_ref, bq2a_ref, Wq2b_ref, bq2b_ref,
          q1_ref, q2_ref):
    acc = jnp.dot(s_ref[0], W1_ref[0:_OBS, :])
    acc += jnp.dot(s_ref[1], W1_ref[_OBS:2 * _OBS, :])
    acc += jnp.dot(s_ref[2], W1_ref[2 * _OBS:3 * _OBS, :])
    off = 3 * _OBS
    acc += jnp.dot(a_ref[0], W1_ref[off:off + _ACT, :])
    acc += jnp.dot(a_ref[1], W1_ref[off + _ACT:off + 2 * _ACT, :])
    acc += jnp.dot(a_ref[2], W1_ref[off + 2 * _ACT:off + 3 * _ACT, :])
    fc1 = jnp.maximum(acc + b1_ref[:], 0.0)

    xw = jnp.dot(fc1, Wg_ref[:])
    clique = (xw[0:1, :] + xw[1:2, :] + xw[2:3, :]) * (1.0 / 3.0)
    row = jax.lax.broadcasted_iota(jnp.int32, (_BLOCK, 1), 0)
    in_clique = jnp.logical_and(pl.program_id(0) == 0, row < 3)
    xw = jnp.where(in_clique, clique, xw)
    g = jnp.maximum(xw + bg_ref[:], 0.0) + fc1

    fc2 = jnp.maximum(jnp.dot(g, W2_ref[:]) + b2_ref[:], 0.0)

    h1 = jnp.maximum(jnp.dot(fc2, Wq1a_ref[:]) + bq1a_ref[:], 0.0)
    q1_ref[:] = jnp.dot(h1, Wq1b_ref[:]) + bq1b_ref[:]
    h2 = jnp.maximum(jnp.dot(fc2, Wq2a_ref[:]) + bq2a_ref[:], 0.0)
    q2_ref[:] = jnp.dot(h2, Wq2b_ref[:]) + bq2b_ref[:]


@functools.partial(jax.jit, static_argnames=("interpret",))
def kernel(s, a, W1, b1, Wg, bg, W2, b2, Wq1a, bq1a, Wq1b, bq1b,
           Wq2a, bq2a, Wq2b, bq2b, interpret=False):
    na, n, obs = s.shape
    h = W1.shape[1]
    grid = (n // _BLOCK,)

    def rows(i):
        return (0, i, 0)

    def full(i):
        return (0, 0)

    in_specs = [
        pl.BlockSpec((na, _BLOCK, obs), rows),
        pl.BlockSpec((na, _BLOCK, a.shape[2]), rows),
        pl.BlockSpec(W1.shape, full),
        pl.BlockSpec((1, h), full),
        pl.BlockSpec(Wg.shape, full),
        pl.BlockSpec((1, h), full),
        pl.BlockSpec(W2.shape, full),
        pl.BlockSpec((1, h), full),
        pl.BlockSpec(Wq1a.shape, full),
        pl.BlockSpec((1, h), full),
        pl.BlockSpec(Wq1b.shape, full),
        pl.BlockSpec((1, 1), full),
        pl.BlockSpec(Wq2a.shape, full),
        pl.BlockSpec((1, h), full),
        pl.BlockSpec(Wq2b.shape, full),
        pl.BlockSpec((1, 1), full),
    ]
    out_specs = [
        pl.BlockSpec((_BLOCK, 1), lambda i: (i, 0)),
        pl.BlockSpec((_BLOCK, 1), lambda i: (i, 0)),
    ]
    q1, q2 = pl.pallas_call(
        _body,
        grid=grid,
        in_specs=in_specs,
        out_specs=out_specs,
        out_shape=[jax.ShapeDtypeStruct((n, 1), jnp.float32)] * 2,
        compiler_params=pltpu.CompilerParams(
            dimension_semantics=("parallel",)),
        interpret=interpret,
    )(s, a, W1, b1.reshape(1, h), Wg, bg.reshape(1, h), W2, b2.reshape(1, h),
      Wq1a, bq1a.reshape(1, h), Wq1b, bq1b.reshape(1, 1),
      Wq2a, bq2a.reshape(1, h), Wq2b, bq2b.reshape(1, 1))
    return (q1, q2)
